# baseline (speedup 1.0000x reference)
import jax
import jax.numpy as jnp
from jax.experimental import pallas as pl
from jax.experimental.pallas import tpu as pltpu

N_WAY, N_SUPPORT, EMB = 20, 5, 128
B, NBOX = 16, 8732
R = B * NBOX
GRID = 59
CA = R // GRID
LROWS = CA * 4 // 128


def _tc_body(t_ref, cls_ref, vec_ref, np_ref, acc_vec, acc_smem):
    i = pl.program_id(0)

    @pl.when(i == 0)
    def _init():
        acc_vec[...] = jnp.zeros_like(acc_vec)
        acc_smem[0] = 0.0

    posf = (t_ref[0] > 0).astype(jnp.float32)
    acc_vec[...] += jax.lax.dot(posf, cls_ref[0])
    acc_smem[0] += jnp.sum(posf)

    @pl.when(i == pl.num_programs(0) - 1)
    def _fin():
        vec_ref[...] = acc_vec[...]
        np_ref[...] = jnp.full((1, 1), acc_smem[0], dtype=jnp.float32)


def _tc_cls(cls_targets, cls_preds):
    t3 = cls_targets.reshape(B, 1, NBOX)
    return pl.pallas_call(
        _tc_body,
        grid=(B,),
        in_specs=[
            pl.BlockSpec((1, 1, NBOX), lambda i: (i, 0, 0)),
            pl.BlockSpec((1, NBOX, EMB), lambda i: (i, 0, 0)),
        ],
        out_specs=[
            pl.BlockSpec((1, EMB), lambda i: (0, 0)),
            pl.BlockSpec((1, 1), lambda i: (0, 0)),
        ],
        out_shape=[
            jax.ShapeDtypeStruct((1, EMB), jnp.float32),
            jax.ShapeDtypeStruct((1, 1), jnp.float32),
        ],
        scratch_shapes=[
            pltpu.VMEM((1, EMB), jnp.float32),
            pltpu.SMEM((2,), jnp.float32),
        ],
        compiler_params=pltpu.CompilerParams(
            dimension_semantics=("arbitrary",),
        ),
    )(t3, cls_preds)


def _loc_body(t2_ref, lp_ref, lt_ref, out_ref, acc_smem):
    i = pl.program_id(0)

    @pl.when(i == 0)
    def _init():
        acc_smem[0] = 0.0

    posf2 = (t2_ref[0] > 0).astype(jnp.float32)
    lane = jax.lax.broadcasted_iota(jnp.int32, (32, 128), 1)
    grp = jax.lax.broadcasted_iota(jnp.int32, (32, 128), 0)
    expand = (lane // 4 == grp).astype(jnp.float32)
    posf4 = jax.lax.dot(posf2, expand)

    diff = lp_ref[0] - lt_ref[0]
    a = jnp.abs(diff)
    sl1 = jnp.where(a < 1.0, 0.5 * diff * diff, a - 0.5)
    acc_smem[0] += jnp.sum(sl1 * posf4)

    @pl.when(i == pl.num_programs(0) - 1)
    def _fin():
        out_ref[...] = jnp.full((1, 1), acc_smem[0], dtype=jnp.float32)


def _tc_loc(loc_preds, loc_targets, cls_targets):
    t2 = cls_targets.reshape(GRID, LROWS, 32)
    lp = loc_preds.reshape(GRID, LROWS, 128)
    lt = loc_targets.reshape(GRID, LROWS, 128)
    return pl.pallas_call(
        _loc_body,
        grid=(GRID,),
        in_specs=[
            pl.BlockSpec((1, LROWS, 32), lambda i: (i, 0, 0)),
            pl.BlockSpec((1, LROWS, 128), lambda i: (i, 0, 0)),
            pl.BlockSpec((1, LROWS, 128), lambda i: (i, 0, 0)),
        ],
        out_specs=pl.BlockSpec((1, 1), lambda i: (0, 0)),
        out_shape=jax.ShapeDtypeStruct((1, 1), jnp.float32),
        scratch_shapes=[pltpu.SMEM((2,), jnp.float32)],
        compiler_params=pltpu.CompilerParams(
            dimension_semantics=("arbitrary",),
        ),
    )(t2, lp, lt)


def _comb_body(vec_ref, np_ref, ll_ref, sup_ref, out_ref):
    num_pos = jnp.sum(np_ref[...])
    loc_loss = jnp.sum(ll_ref[...])
    mean_q = vec_ref[...] / num_pos
    protos = (sup_ref[:, 0, :] + sup_ref[:, 1, :] + sup_ref[:, 2, :]
              + sup_ref[:, 3, :] + sup_ref[:, 4, :]) * (1.0 / N_SUPPORT)
    d = jnp.sum((mean_q - protos) ** 2, axis=1)
    neg = -d
    m = jnp.max(neg)
    lse = m + jnp.log(jnp.sum(jnp.exp(neg - m)))
    cls_loss = lse - neg[0]
    out_ref[...] = jnp.full((1, 1), cls_loss + loc_loss / num_pos,
                            dtype=jnp.float32)


def kernel(loc_preds, loc_targets, cls_preds, cls_targets, supports):
    loc_part = _tc_loc(loc_preds, loc_targets, cls_targets)
    vec, npos = _tc_cls(cls_targets, cls_preds)
    out = pl.pallas_call(
        _comb_body,
        out_shape=jax.ShapeDtypeStruct((1, 1), jnp.float32),
    )(vec, npos, loc_part, supports)
    return out[0, 0]
